# baseline (speedup 1.0000x reference)
import functools

import jax
import jax.numpy as jnp
from jax import lax
from jax.experimental import pallas as pl
from jax.experimental.pallas import tpu as pltpu
from jax.experimental.pallas import tpu_sc as plsc

_NCTX = 200
_NNEXT = 4096
_NNEG = 16384
_NROWS = _NCTX + _NNEXT + _NNEG
_FEAT = 64
_B = 21504
_NW = 32
_BPW = _B // _NW
_CHUNKS = [(o, min(128, _BPW - o)) for o in range(0, _BPW, 128)]

_mesh = plsc.VectorSubcoreMesh(core_axis_name="c", subcore_axis_name="s")


@functools.partial(
    pl.kernel,
    mesh=_mesh,
    out_type=jax.ShapeDtypeStruct((_B // 2, 2 * _FEAT), jnp.bfloat16),
    scratch_types=[
        pltpu.VMEM((_BPW,), jnp.int32),
        pltpu.VMEM((_BPW, _FEAT), jnp.bfloat16),
        pltpu.SemaphoreType.DMA,
        pltpu.SemaphoreType.DMA,
    ],
    compiler_params=pltpu.CompilerParams(use_tc_tiling_on_sc=False),
)
def _sc_gather1(tab, idx_hbm, out, idx_v, rows_v, sem, osem):
    wid = lax.axis_index("s") * 2 + lax.axis_index("c")
    base = wid * _BPW
    half = _BPW // 2
    pbase = wid * half
    pltpu.sync_copy(idx_hbm.at[pl.ds(base, _BPW)], idx_v)
    cps = [pltpu.async_copy(tab.at[idx_v.at[pl.ds(o, n)]],
                            rows_v.at[pl.ds(o, n)], sem)
           for o, n in _CHUNKS]
    for cp in cps:
        cp.wait()
    cp1 = pltpu.async_copy(rows_v.at[pl.ds(0, half)],
                           out.at[pl.ds(pbase, half), pl.ds(0, _FEAT)],
                           osem)
    cp2 = pltpu.async_copy(rows_v.at[pl.ds(half, half)],
                           out.at[pl.ds(pbase, half), pl.ds(_FEAT, _FEAT)],
                           osem)
    cp1.wait()
    cp2.wait()


_RBP = _BPW // 2
_CTX2 = 256


def _tc_body(et_, ea_, er_, ct_, ca_, cr_, aff, nrm):
    dn = (((1,), (1,)), ((), ()))
    neg_inf = jnp.float32(-jnp.inf)
    col = lax.broadcasted_iota(jnp.int32, (_RBP, _CTX2), 1)
    valid = col < _NCTX

    se = jnp.full((_RBP, _CTX2), 0.0, jnp.float32)
    so = jnp.full((_RBP, _CTX2), 0.0, jnp.float32)
    nrm_e = jnp.zeros((_RBP,), jnp.float32)
    nrm_o = jnp.zeros((_RBP,), jnp.float32)
    for eref, cref in ((et_, ct_), (ea_, ca_), (er_, cr_)):
        eb = eref[...]
        cv = cref[...][:, :_FEAT]
        ev, od = eb[:, :_FEAT], eb[:, _FEAT:]
        se += lax.dot_general(ev, cv, dn,
                              preferred_element_type=jnp.float32)
        so += lax.dot_general(od, cv, dn,
                              preferred_element_type=jnp.float32)
        ev32 = ev.astype(jnp.float32)
        od32 = od.astype(jnp.float32)
        nrm_e += jnp.sum(ev32 * ev32, 1)
        nrm_o += jnp.sum(od32 * od32, 1)

    se = jnp.where(valid, se, neg_inf)
    so = jnp.where(valid, so, neg_inf)
    aff[...] = jnp.stack([jnp.max(se, 1), jnp.max(so, 1)])[None]
    nrm[...] = jnp.stack([jnp.sqrt(nrm_e), jnp.sqrt(nrm_o)])[None]


_eb = pl.BlockSpec((_RBP, 2 * _FEAT), lambda i: (i, 0))
_cb = pl.BlockSpec((_CTX2, 2 * _FEAT), lambda i: (0, 0))
_ob = pl.BlockSpec((1, 2, _RBP), lambda i: (i, 0, 0))

_tc_compute = pl.pallas_call(
    _tc_body,
    grid=(_NW,),
    in_specs=[_eb, _eb, _eb, _cb, _cb, _cb],
    out_specs=[_ob, _ob],
    out_shape=[jax.ShapeDtypeStruct((_NW, 2, _RBP), jnp.float32)] * 2,
)


def kernel(track_context, album_context, artist_context,
           next_track, next_album, next_artist,
           neg_track, neg_album, neg_artist,
           track_table, album_table, artist_table):
    pad = jnp.arange(_B - _NROWS, dtype=jnp.int32)
    idx_t = jnp.concatenate([track_context.astype(jnp.int32),
                             next_track.astype(jnp.int32),
                             neg_track.astype(jnp.int32), pad])
    idx_a = jnp.concatenate([album_context.astype(jnp.int32),
                             next_album.astype(jnp.int32),
                             neg_album.astype(jnp.int32), pad])
    idx_r = jnp.concatenate([artist_context.astype(jnp.int32),
                             next_artist.astype(jnp.int32),
                             neg_artist.astype(jnp.int32), pad])
    p_t = _sc_gather1(track_table.astype(jnp.bfloat16), idx_t)
    p_a = _sc_gather1(album_table.astype(jnp.bfloat16), idx_a)
    p_r = _sc_gather1(artist_table.astype(jnp.bfloat16), idx_r)
    aff3, nrm3 = _tc_compute(p_t, p_a, p_r, p_t, p_a, p_r)
    aff = aff3.reshape(_B)
    nrm = nrm3.reshape(_B)
    return (aff[_NCTX:_NCTX + _NNEXT],
            aff[_NCTX + _NNEXT:_NROWS],
            nrm[:_NROWS])


# speedup vs baseline: 1.7087x; 1.7087x over previous
import functools

import jax
import jax.numpy as jnp
from jax import lax
from jax.experimental import pallas as pl
from jax.experimental.pallas import tpu as pltpu
from jax.experimental.pallas import tpu_sc as plsc

_NCTX = 200
_NNEXT = 4096
_NNEG = 16384
_NROWS = _NCTX + _NNEXT + _NNEG
_FEAT = 64
_B = 21504
_NW = 32
_BPW = _B // _NW
_CHUNKS = [(o, min(128, _BPW - o)) for o in range(0, _BPW, 128)]

_mesh = plsc.VectorSubcoreMesh(core_axis_name="c", subcore_axis_name="s")


@functools.partial(
    pl.kernel,
    mesh=_mesh,
    out_type=jax.ShapeDtypeStruct((_B // 2, 2 * _FEAT), jnp.float32),
    scratch_types=[
        pltpu.VMEM((_BPW,), jnp.int32),
        pltpu.VMEM((_BPW, _FEAT), jnp.float32),
        pltpu.SemaphoreType.DMA,
        pltpu.SemaphoreType.DMA,
    ],
    compiler_params=pltpu.CompilerParams(use_tc_tiling_on_sc=False),
)
def _sc_gather1(tab, idx_hbm, out, idx_v, rows_v, sem, osem):
    wid = lax.axis_index("s") * 2 + lax.axis_index("c")
    base = wid * _BPW
    half = _BPW // 2
    pbase = wid * half
    pltpu.sync_copy(idx_hbm.at[pl.ds(base, _BPW)], idx_v)
    cps = [pltpu.async_copy(tab.at[idx_v.at[pl.ds(o, n)]],
                            rows_v.at[pl.ds(o, n)], sem)
           for o, n in _CHUNKS]
    for cp in cps:
        cp.wait()
    cp1 = pltpu.async_copy(rows_v.at[pl.ds(0, half)],
                           out.at[pl.ds(pbase, half), pl.ds(0, _FEAT)],
                           osem)
    cp2 = pltpu.async_copy(rows_v.at[pl.ds(half, half)],
                           out.at[pl.ds(pbase, half), pl.ds(_FEAT, _FEAT)],
                           osem)
    cp1.wait()
    cp2.wait()


_RBP = _BPW // 2
_CTX2 = 256


def _tc_body(et_, ea_, er_, ct_, ca_, cr_, aff, nrm):
    dn = (((1,), (1,)), ((), ()))
    neg_inf = jnp.float32(-jnp.inf)
    col = lax.broadcasted_iota(jnp.int32, (_RBP, _CTX2), 1)
    valid = col < _NCTX

    se = jnp.full((_RBP, _CTX2), 0.0, jnp.float32)
    so = jnp.full((_RBP, _CTX2), 0.0, jnp.float32)
    nrm_e = jnp.zeros((_RBP,), jnp.float32)
    nrm_o = jnp.zeros((_RBP,), jnp.float32)
    for eref, cref in ((et_, ct_), (ea_, ca_), (er_, cr_)):
        eb = eref[...]
        cv = cref[...][:, :_FEAT]
        ev, od = eb[:, :_FEAT], eb[:, _FEAT:]
        se += lax.dot_general(ev, cv, dn)
        so += lax.dot_general(od, cv, dn)
        nrm_e += jnp.sum(ev * ev, 1)
        nrm_o += jnp.sum(od * od, 1)

    se = jnp.where(valid, se, neg_inf)
    so = jnp.where(valid, so, neg_inf)
    aff[...] = jnp.stack([jnp.max(se, 1), jnp.max(so, 1)])[None]
    nrm[...] = jnp.stack([jnp.sqrt(nrm_e), jnp.sqrt(nrm_o)])[None]


_eb = pl.BlockSpec((_RBP, 2 * _FEAT), lambda i: (i, 0))
_cb = pl.BlockSpec((_CTX2, 2 * _FEAT), lambda i: (0, 0))
_ob = pl.BlockSpec((1, 2, _RBP), lambda i: (i, 0, 0))

_tc_compute = pl.pallas_call(
    _tc_body,
    grid=(_NW,),
    in_specs=[_eb, _eb, _eb, _cb, _cb, _cb],
    out_specs=[_ob, _ob],
    out_shape=[jax.ShapeDtypeStruct((_NW, 2, _RBP), jnp.float32)] * 2,
)


def kernel(track_context, album_context, artist_context,
           next_track, next_album, next_artist,
           neg_track, neg_album, neg_artist,
           track_table, album_table, artist_table):
    pad = jnp.arange(_B - _NROWS, dtype=jnp.int32)
    idx_t = jnp.concatenate([track_context.astype(jnp.int32),
                             next_track.astype(jnp.int32),
                             neg_track.astype(jnp.int32), pad])
    idx_a = jnp.concatenate([album_context.astype(jnp.int32),
                             next_album.astype(jnp.int32),
                             neg_album.astype(jnp.int32), pad])
    idx_r = jnp.concatenate([artist_context.astype(jnp.int32),
                             next_artist.astype(jnp.int32),
                             neg_artist.astype(jnp.int32), pad])
    p_t = _sc_gather1(track_table, idx_t)
    p_a = _sc_gather1(album_table, idx_a)
    p_r = _sc_gather1(artist_table, idx_r)
    aff3, nrm3 = _tc_compute(p_t, p_a, p_r, p_t, p_a, p_r)
    aff = aff3.reshape(_B)
    nrm = nrm3.reshape(_B)
    return (aff[_NCTX:_NCTX + _NNEXT],
            aff[_NCTX + _NNEXT:_NROWS],
            nrm[:_NROWS])
